# Initial kernel scaffold; baseline (speedup 1.0000x reference)
import jax
import jax.numpy as jnp
from jax.experimental import pallas as pl


def kernel(aspect_ids, class_ids, table):
    raise NotImplementedError("write your pallas kernel here")


# speedup vs baseline: 3.3955x; 3.3955x over previous
import functools

import jax
import jax.numpy as jnp
from jax import lax
from jax.experimental import pallas as pl
from jax.experimental.pallas import tpu as pltpu
from jax.experimental.pallas import tpu_sc as plsc

N_ASPECTS = 100000
N_CLASSES = 4
D = 128
B = 4096
H = 50

NC = 2
NS = 16
NW = NC * NS
RPW = B // NW
CH = 8
NCHUNK = RPW // CH
IDS_PER_CHUNK = CH * H
SUB = 80
NSUB = IDS_PER_CHUNK // SUB
NVD = D // 16


def _body(aspect_hbm, class_hbm, table_hbm, out_hbm,
          av, cv, cntv, t0, rows0, rows1, ob0, ob1,
          gsem0, gsem1, osem0, osem1):
    wid = lax.axis_index("s") * NC + lax.axis_index("c")
    base = wid * RPW
    fbase = base * H

    pltpu.sync_copy(aspect_hbm.at[pl.ds(fbase, RPW * H)], av)
    pltpu.sync_copy(class_hbm.at[pl.ds(fbase, RPW * H)], cv)
    pltpu.sync_copy(table_hbm.at[pl.ds(0, 1)], t0)

    @pl.loop(0, RPW * H // 16)
    def _ids(i):
        off = pl.multiple_of(i * 16, 16)
        a = av[pl.ds(off, 16)]
        c = cv[pl.ds(off, 16)]
        idv = a * N_CLASSES + c
        av[pl.ds(off, 16)] = jnp.where(idv != 0, idv + 1, idv)

    lanes = lax.iota(jnp.int32, 16)

    @pl.loop(0, RPW // 16)
    def _cnt(g):
        rowv = g * 16 + lanes

        def cbody(l, acc):
            idv = plsc.load_gather(av, [rowv * H + l])
            return acc + jnp.where(idv == 0, 1.0, 0.0).astype(jnp.float32)

        acc = lax.fori_loop(0, H, cbody, jnp.zeros((16,), jnp.float32))
        cntv[pl.ds(pl.multiple_of(g * 16, 16), 16)] = acc

    def fire(c, rbuf, sem):
        for j in range(NSUB):
            off = pl.multiple_of(c * IDS_PER_CHUNK + j * SUB, 8)
            pltpu.async_copy(
                table_hbm.at[av.at[pl.ds(off, SUB)]],
                rbuf.at[pl.ds(j * SUB, SUB)], sem)

    def drain(rbuf, sem):
        for j in range(NSUB):
            pltpu.make_async_copy(
                table_hbm.at[av.at[pl.ds(j * SUB, SUB)]],
                rbuf.at[pl.ds(j * SUB, SUB)], sem).wait()

    def accum(c, rbuf, obuf):
        for b in range(CH):
            r = c * CH + b
            goff = pl.multiple_of((r // 16) * 16, 16)
            cgrp = cntv[pl.ds(goff, 16)]
            pad = jnp.sum(jnp.where(lanes == r % 16, cgrp, 0.0))
            padv = jnp.broadcast_to(pad, (16,))
            invv = 1.0 / (jnp.float32(H) - padv)

            def abody(l, accs):
                row = b * H + l
                return tuple(accs[v] + rbuf[row, pl.ds(v * 16, 16)]
                             for v in range(NVD))

            accs = lax.fori_loop(
                0, H, abody,
                tuple(jnp.zeros((16,), jnp.float32) for _ in range(NVD)))
            for v in range(NVD):
                t0s = t0[0, pl.ds(v * 16, 16)]
                obuf[b, pl.ds(v * 16, 16)] = (accs[v] - padv * t0s) * invv

    def out_dma(c, obuf, osem):
        ooff = pl.multiple_of(base + c * CH, 8)
        pltpu.async_copy(obuf, out_hbm.at[pl.ds(ooff, CH)], osem)

    def out_wait(obuf, osem):
        pltpu.make_async_copy(obuf, out_hbm.at[pl.ds(0, CH)], osem).wait()

    @pl.loop(0, NCHUNK // 2)
    def _steps(s):
        c0 = s * 2
        c1 = c0 + 1
        fire(c0, rows0, gsem0)
        fire(c1, rows1, gsem1)
        drain(rows0, gsem0)
        accum(c0, rows0, ob0)
        out_dma(c0, ob0, osem0)
        drain(rows1, gsem1)
        accum(c1, rows1, ob1)
        out_dma(c1, ob1, osem1)
        out_wait(ob0, osem0)
        out_wait(ob1, osem1)


@jax.jit
def _run(aspect_flat, class_flat, table):
    mesh = plsc.VectorSubcoreMesh(core_axis_name="c", subcore_axis_name="s")
    k = pl.kernel(
        _body,
        out_type=jax.ShapeDtypeStruct((B, D), jnp.float32),
        mesh=mesh,
        compiler_params=pltpu.CompilerParams(needs_layout_passes=False),
        scratch_types=[
            pltpu.VMEM((RPW * H,), jnp.int32),
            pltpu.VMEM((RPW * H,), jnp.int32),
            pltpu.VMEM((RPW,), jnp.float32),
            pltpu.VMEM((1, D), jnp.float32),
            pltpu.VMEM((IDS_PER_CHUNK, D), jnp.float32),
            pltpu.VMEM((IDS_PER_CHUNK, D), jnp.float32),
            pltpu.VMEM((CH, D), jnp.float32),
            pltpu.VMEM((CH, D), jnp.float32),
            pltpu.SemaphoreType.DMA,
            pltpu.SemaphoreType.DMA,
            pltpu.SemaphoreType.DMA,
            pltpu.SemaphoreType.DMA,
        ],
    )
    return k(aspect_flat, class_flat, table)


def kernel(aspect_ids, class_ids, table):
    aspect_flat = aspect_ids.astype(jnp.int32).reshape(-1)
    class_flat = class_ids.astype(jnp.int32).reshape(-1)
    return _run(aspect_flat, class_flat, table.astype(jnp.float32))


# speedup vs baseline: 4.7562x; 1.4008x over previous
import functools

import jax
import jax.numpy as jnp
from jax import lax
from jax.experimental import pallas as pl
from jax.experimental.pallas import tpu as pltpu
from jax.experimental.pallas import tpu_sc as plsc

N_ASPECTS = 100000
N_CLASSES = 4
D = 128
B = 4096
H = 50

NC = 2
NS = 16
NW = NC * NS
RPW = B // NW
CH = 8
NCHUNK = RPW // CH
IDS_PER_CHUNK = CH * H
SUB = 80
NSUB = IDS_PER_CHUNK // SUB
NVD = D // 16


def _body(aspect_hbm, class_hbm, table_hbm, out_hbm,
          av, cv, cntv, t0, rows0, rows1, ob0, ob1,
          gsem0, gsem1, osem0, osem1):
    wid = lax.axis_index("s") * NC + lax.axis_index("c")
    base = wid * RPW
    fbase = base * H

    pltpu.sync_copy(aspect_hbm.at[pl.ds(fbase, RPW * H)], av)
    pltpu.sync_copy(class_hbm.at[pl.ds(fbase, RPW * H)], cv)
    pltpu.sync_copy(table_hbm.at[pl.ds(0, 1)], t0)

    @pl.loop(0, RPW * H // 16)
    def _ids(i):
        off = pl.multiple_of(i * 16, 16)
        a = av[pl.ds(off, 16)]
        c = cv[pl.ds(off, 16)]
        idv = a * N_CLASSES + c
        av[pl.ds(off, 16)] = jnp.where(idv != 0, idv + 1, idv)

    lanes = lax.iota(jnp.int32, 16)

    def fire(c, rbuf, sem):
        for j in range(NSUB):
            off = pl.multiple_of(c * IDS_PER_CHUNK + j * SUB, 8)
            pltpu.async_copy(
                table_hbm.at[av.at[pl.ds(off, SUB)]],
                rbuf.at[pl.ds(j * SUB, SUB)], sem)

    def drain(rbuf, sem):
        for j in range(NSUB):
            pltpu.make_async_copy(
                table_hbm.at[av.at[pl.ds(j * SUB, SUB)]],
                rbuf.at[pl.ds(j * SUB, SUB)], sem).wait()

    def accum(c, rbuf, obuf):
        for b in range(CH):
            r = c * CH + b
            goff = pl.multiple_of((r // 16) * 16, 16)
            cgrp = cntv[pl.ds(goff, 16)]
            pad = jnp.sum(jnp.where(lanes == r % 16, cgrp, 0.0))
            padv = jnp.broadcast_to(pad, (16,))
            invv = 1.0 / (jnp.float32(H) - padv)

            def abody(l, accs):
                row = b * H + l
                return tuple(accs[v] + rbuf[row, pl.ds(v * 16, 16)]
                             for v in range(NVD))

            accs = lax.fori_loop(
                0, H, abody,
                tuple(jnp.zeros((16,), jnp.float32) for _ in range(NVD)))
            for v in range(NVD):
                t0s = t0[0, pl.ds(v * 16, 16)]
                obuf[b, pl.ds(v * 16, 16)] = (accs[v] - padv * t0s) * invv

    def out_dma(c, obuf, osem):
        ooff = pl.multiple_of(base + c * CH, 8)
        pltpu.async_copy(obuf, out_hbm.at[pl.ds(ooff, CH)], osem)

    def out_wait(obuf, osem):
        pltpu.make_async_copy(obuf, out_hbm.at[pl.ds(0, CH)], osem).wait()

    fire(0, rows0, gsem0)
    fire(1, rows1, gsem1)

    @pl.loop(0, RPW // 16)
    def _cnt(g):
        rowv = g * 16 + lanes

        def cbody(l, acc):
            idv = plsc.load_gather(av, [rowv * H + l])
            return acc + jnp.where(idv == 0, 1.0, 0.0).astype(jnp.float32)

        acc = lax.fori_loop(0, H, cbody, jnp.zeros((16,), jnp.float32))
        cntv[pl.ds(pl.multiple_of(g * 16, 16), 16)] = acc

    drain(rows0, gsem0)
    accum(0, rows0, ob0)
    fire(2, rows0, gsem0)
    out_dma(0, ob0, osem0)
    drain(rows1, gsem1)
    accum(1, rows1, ob1)
    fire(3, rows1, gsem1)
    out_dma(1, ob1, osem1)

    @pl.loop(1, NCHUNK // 2 - 1)
    def _steps(s):
        c0 = s * 2
        c1 = c0 + 1
        drain(rows0, gsem0)
        out_wait(ob0, osem0)
        accum(c0, rows0, ob0)
        fire(c0 + 2, rows0, gsem0)
        out_dma(c0, ob0, osem0)
        drain(rows1, gsem1)
        out_wait(ob1, osem1)
        accum(c1, rows1, ob1)
        fire(c1 + 2, rows1, gsem1)
        out_dma(c1, ob1, osem1)

    drain(rows0, gsem0)
    out_wait(ob0, osem0)
    accum(NCHUNK - 2, rows0, ob0)
    out_dma(NCHUNK - 2, ob0, osem0)
    drain(rows1, gsem1)
    out_wait(ob1, osem1)
    accum(NCHUNK - 1, rows1, ob1)
    out_dma(NCHUNK - 1, ob1, osem1)
    out_wait(ob0, osem0)
    out_wait(ob1, osem1)


@jax.jit
def _run(aspect_flat, class_flat, table):
    mesh = plsc.VectorSubcoreMesh(core_axis_name="c", subcore_axis_name="s")
    k = pl.kernel(
        _body,
        out_type=jax.ShapeDtypeStruct((B, D), jnp.float32),
        mesh=mesh,
        compiler_params=pltpu.CompilerParams(needs_layout_passes=False),
        scratch_types=[
            pltpu.VMEM((RPW * H,), jnp.int32),
            pltpu.VMEM((RPW * H,), jnp.int32),
            pltpu.VMEM((RPW,), jnp.float32),
            pltpu.VMEM((1, D), jnp.float32),
            pltpu.VMEM((IDS_PER_CHUNK, D), jnp.float32),
            pltpu.VMEM((IDS_PER_CHUNK, D), jnp.float32),
            pltpu.VMEM((CH, D), jnp.float32),
            pltpu.VMEM((CH, D), jnp.float32),
            pltpu.SemaphoreType.DMA,
            pltpu.SemaphoreType.DMA,
            pltpu.SemaphoreType.DMA,
            pltpu.SemaphoreType.DMA,
        ],
    )
    return k(aspect_flat, class_flat, table)


def kernel(aspect_ids, class_ids, table):
    aspect_flat = aspect_ids.astype(jnp.int32).reshape(-1)
    class_flat = class_ids.astype(jnp.int32).reshape(-1)
    return _run(aspect_flat, class_flat, table.astype(jnp.float32))
